# baseline (speedup 1.0000x reference)
import functools

import jax
import jax.numpy as jnp
from jax import lax
from jax.experimental import pallas as pl
from jax.experimental.pallas import tpu as pltpu
from jax.experimental.pallas import tpu_sc as plsc

_NC = 2
_NS = 16
_NW = _NC * _NS


def _table_body(emb_ref, wt_ref, b_ref, out_ref):
    out_ref[...] = (
        jax.lax.dot_general(
            emb_ref[...],
            wt_ref[...],
            (((1,), (0,)), ((), ())),
            preferred_element_type=jnp.float32,
            precision=jax.lax.Precision.HIGHEST,
        )
        + b_ref[...]
    )


def _build_table_padded(emb, Wt, b2d, vp):
    v, _ = emb.shape
    return pl.pallas_call(
        _table_body,
        out_shape=jax.ShapeDtypeStruct((v, vp), jnp.float32),
    )(emb, Wt, b2d)


def _make_gather(vp, n_rows, chunk):
    per_w = n_rows // _NW
    n_chunks = per_w // chunk
    mesh = plsc.VectorSubcoreMesh(core_axis_name="c", subcore_axis_name="s")

    @functools.partial(
        pl.kernel,
        mesh=mesh,
        out_type=jax.ShapeDtypeStruct((n_rows, vp), jnp.float32),
        scratch_types=[
            pltpu.VMEM((per_w,), jnp.int32),
            pltpu.VMEM((chunk, vp), jnp.float32),
            pltpu.SemaphoreType.DMA,
        ],
        compiler_params=pltpu.CompilerParams(use_tc_tiling_on_sc=False),
    )
    def gather(table_hbm, idx_hbm, out_hbm, idx_v, rows_v, sem):
        wid = lax.axis_index("s") * _NC + lax.axis_index("c")
        base = wid * per_w
        pltpu.sync_copy(idx_hbm.at[pl.ds(base, per_w)], idx_v)

        def body(g, carry):
            off = g * chunk
            pltpu.async_copy(
                table_hbm.at[idx_v.at[pl.ds(off, chunk)]], rows_v, sem
            ).wait()
            pltpu.sync_copy(rows_v, out_hbm.at[pl.ds(base + off, chunk)])
            return carry

        lax.fori_loop(0, n_chunks, body, 0)

    return gather


def kernel(x, emb, W, b):
    bsz, seq = x.shape
    v, _ = emb.shape
    table = _build_table_padded(emb, W.T, b.reshape(1, v), v)
    flat_idx = x.reshape(-1).astype(jnp.int32)
    out = _make_gather(v, bsz * seq, 40)(table, flat_idx)
    return out.reshape(bsz, seq, v)


# speedup vs baseline: 1.0157x; 1.0157x over previous
import functools

import jax
import jax.numpy as jnp
from jax import lax
from jax.experimental import pallas as pl
from jax.experimental.pallas import tpu as pltpu
from jax.experimental.pallas import tpu_sc as plsc

_NC = 2
_NS = 16
_NW = _NC * _NS


def _table_body(emb_ref, wt_ref, b_ref, out_ref):
    out_ref[...] = (
        jax.lax.dot_general(
            emb_ref[...],
            wt_ref[...],
            (((1,), (0,)), ((), ())),
            preferred_element_type=jnp.float32,
            precision=jax.lax.Precision.HIGHEST,
        )
        + b_ref[...]
    )


def _build_table(emb, Wt, b2d):
    v, _ = emb.shape
    return pl.pallas_call(
        _table_body,
        out_shape=jax.ShapeDtypeStruct((v, v), jnp.float32),
    )(emb, Wt, b2d)


def _make_gather(bsz, seq, v, seq_p):
    per_w = bsz // _NW
    mesh = plsc.VectorSubcoreMesh(core_axis_name="c", subcore_axis_name="s")

    @functools.partial(
        pl.kernel,
        mesh=mesh,
        out_type=jax.ShapeDtypeStruct((bsz, seq, v), jnp.float32),
        scratch_types=[
            pltpu.VMEM((per_w * seq_p,), jnp.int32),
            pltpu.VMEM((seq, v), jnp.float32),
            pltpu.SemaphoreType.DMA,
        ],
        compiler_params=pltpu.CompilerParams(use_tc_tiling_on_sc=False),
    )
    def gather(table_hbm, idx_hbm, out_hbm, idx_v, rows_v, sem):
        wid = lax.axis_index("s") * _NC + lax.axis_index("c")
        b0 = wid * per_w
        pltpu.sync_copy(idx_hbm.at[pl.ds(b0 * seq_p, per_w * seq_p)], idx_v)

        def body(g, carry):
            pltpu.async_copy(
                table_hbm.at[idx_v.at[pl.ds(g * seq_p, seq)]], rows_v, sem
            ).wait()
            pltpu.sync_copy(rows_v, out_hbm.at[b0 + g])
            return carry

        lax.fori_loop(0, per_w, body, 0)

    return gather


def kernel(x, emb, W, b):
    bsz, seq = x.shape
    v, _ = emb.shape
    seq_p = (seq + 7) // 8 * 8
    table = _build_table(emb, W.T, b.reshape(1, v))
    idx_p = jnp.pad(x.astype(jnp.int32), ((0, 0), (0, seq_p - seq))).reshape(-1)
    out = _make_gather(bsz, seq, v, seq_p)(table, idx_p)
    return out


# speedup vs baseline: 1.4618x; 1.4392x over previous
import functools

import jax
import jax.numpy as jnp
from jax import lax
from jax.experimental import pallas as pl
from jax.experimental.pallas import tpu as pltpu
from jax.experimental.pallas import tpu_sc as plsc

_NC = 2
_NS = 16
_NW = _NC * _NS


def _make_gather(bsz, seq, seq_p, dp):
    per_w = bsz // _NW
    mesh = plsc.VectorSubcoreMesh(core_axis_name="c", subcore_axis_name="s")

    @functools.partial(
        pl.kernel,
        mesh=mesh,
        out_type=jax.ShapeDtypeStruct((bsz, seq, dp), jnp.float32),
        scratch_types=[
            pltpu.VMEM((per_w * seq_p,), jnp.int32),
            pltpu.VMEM((seq, dp), jnp.float32),
            pltpu.SemaphoreType.DMA,
        ],
    )
    def gather(emb_hbm, idx_hbm, h_hbm, idx_v, rows_v, sem):
        wid = lax.axis_index("s") * _NC + lax.axis_index("c")
        b0 = wid * per_w
        pltpu.sync_copy(idx_hbm.at[pl.ds(b0 * seq_p, per_w * seq_p)], idx_v)

        def body(g, carry):
            pltpu.async_copy(
                emb_hbm.at[idx_v.at[pl.ds(g * seq_p, seq)]], rows_v, sem
            ).wait()
            pltpu.sync_copy(rows_v, h_hbm.at[b0 + g])
            return carry

        lax.fori_loop(0, per_w, body, 0)

    return gather


def _proj_body(h_ref, wt_ref, b_ref, out_ref):
    nb, seq, dp = h_ref.shape
    acc = jax.lax.dot_general(
        h_ref[...].reshape(nb * seq, dp),
        wt_ref[...],
        (((1,), (0,)), ((), ())),
        preferred_element_type=jnp.float32,
        precision=jax.lax.Precision.HIGHEST,
    )
    out_ref[...] = acc.reshape(nb, seq, -1) + b_ref[...][None]


def _make_proj(bsz, seq, v, dp, nb):
    grid = (bsz // nb,)
    return pl.pallas_call(
        _proj_body,
        grid=grid,
        in_specs=[
            pl.BlockSpec((nb, seq, dp), lambda g: (g, 0, 0)),
            pl.BlockSpec((dp, v), lambda g: (0, 0)),
            pl.BlockSpec((1, v), lambda g: (0, 0)),
        ],
        out_specs=pl.BlockSpec((nb, seq, v), lambda g: (g, 0, 0)),
        out_shape=jax.ShapeDtypeStruct((bsz, seq, v), jnp.float32),
    )


def kernel(x, emb, W, b):
    bsz, seq = x.shape
    v, d = emb.shape
    dp = (d + 127) // 128 * 128
    seq_p = (seq + 7) // 8 * 8
    emb_p = jnp.pad(emb, ((0, 0), (0, dp - d)))
    wt_p = jnp.pad(W.T, ((0, dp - d), (0, 0)))
    idx_p = jnp.pad(x.astype(jnp.int32), ((0, 0), (0, seq_p - seq))).reshape(-1)
    h = _make_gather(bsz, seq, seq_p, dp)(emb_p, idx_p)
    out = _make_proj(bsz, seq, v, dp, 16)(h, wt_p, b.reshape(1, v))
    return out


# speedup vs baseline: 2.5869x; 1.7697x over previous
import functools

import jax
import jax.numpy as jnp
from jax import lax
from jax.experimental import pallas as pl
from jax.experimental.pallas import tpu as pltpu
from jax.experimental.pallas import tpu_sc as plsc

_NC = 2
_NS = 16
_NW = _NC * _NS


def _make_gather(bsz, seq, dp):
    nb = bsz // _NW
    mesh = plsc.VectorSubcoreMesh(core_axis_name="c", subcore_axis_name="s")

    @functools.partial(
        pl.kernel,
        mesh=mesh,
        out_type=jax.ShapeDtypeStruct((seq, bsz, dp), jnp.float32),
        scratch_types=[
            pltpu.VMEM((seq * nb,), jnp.int32),
            pltpu.VMEM((nb, dp), jnp.float32),
            pltpu.SemaphoreType.DMA,
        ],
    )
    def gather(emb_hbm, idx_hbm, h_hbm, idx_v, rows_v, sem):
        wid = lax.axis_index("s") * _NC + lax.axis_index("c")
        b0 = wid * nb
        def stage(l, carry):
            pltpu.sync_copy(
                idx_hbm.at[pl.ds(l * bsz + b0, nb)], idx_v.at[pl.ds(l * nb, nb)]
            )
            return carry

        lax.fori_loop(0, seq, stage, 0)

        def body(l, carry):
            pltpu.async_copy(
                emb_hbm.at[idx_v.at[pl.ds(l * nb, nb)]], rows_v, sem
            ).wait()
            pltpu.sync_copy(rows_v, h_hbm.at[l, pl.ds(b0, nb)])
            return carry

        lax.fori_loop(0, seq, body, 0)

    return gather


def _proj_body(w_ref, h_ref, b_ref, out_ref):
    acc = jax.lax.dot_general(
        w_ref[...],
        h_ref[0],
        (((1,), (1,)), ((), ())),
        preferred_element_type=jnp.float32,
        precision=jax.lax.Precision.HIGHEST,
    )
    out_ref[...] = (acc + b_ref[...])[None]


def _make_proj(bsz, seq, v, dp):
    return pl.pallas_call(
        _proj_body,
        grid=(seq,),
        in_specs=[
            pl.BlockSpec((v, dp), lambda g: (0, 0)),
            pl.BlockSpec((1, bsz, dp), lambda g: (g, 0, 0)),
            pl.BlockSpec((v, 1), lambda g: (0, 0)),
        ],
        out_specs=pl.BlockSpec((1, v, bsz), lambda g: (g, 0, 0)),
        out_shape=jax.ShapeDtypeStruct((seq, v, bsz), jnp.float32),
    )


def kernel(x, emb, W, b):
    bsz, seq = x.shape
    v, d = emb.shape
    dp = (d + 127) // 128 * 128
    emb_p = jnp.pad(emb, ((0, 0), (0, dp - d)))
    w_p = jnp.pad(W, ((0, 0), (0, dp - d)))
    idx_t = jnp.transpose(x.astype(jnp.int32)).reshape(-1)
    h = _make_gather(bsz, seq, dp)(emb_p, idx_t)
    out_t = _make_proj(bsz, seq, v, dp)(w_p, h, b.reshape(v, 1))
    return jnp.transpose(out_t, (2, 0, 1))
